# baseline (speedup 1.0000x reference)
import functools

import jax
import jax.numpy as jnp
from jax import lax
from jax.experimental import pallas as pl
from jax.experimental.pallas import tpu as pltpu
from jax.experimental.pallas import tpu_sc as plsc

HEADS = 4
H = 1024
V = 32000
B = 128
THRESH = 0.09

VT = 3200
NVT = V // VT
CW = 128
CPT = VT // CW
NCH = V // CW
NCHP = 256
NROWS = HEADS * B
NWORK = 32
RPW = NROWS // NWORK
NEG = -3.0e38


def _tc_body(x_ref, wres_ref, bres_ref, wlm_ref,
             cmax_ref, cam_ref, csum_ref, h_out_ref, h_ref):
    vt = pl.program_id(1)

    @pl.when(vt == 0)
    def _():
        x = x_ref[...]
        pre = lax.dot_general(x, wres_ref[0], (((1,), (1,)), ((), ())),
                              preferred_element_type=jnp.float32)
        pre = pre + bres_ref[0]
        hv = x + pre * jax.nn.sigmoid(pre)
        h_ref[...] = hv
        h_out_ref[0] = hv

    logits = lax.dot_general(h_ref[...], wlm_ref[0], (((1,), (1,)), ((), ())),
                             preferred_element_type=jnp.float32)

    mx_cols = []
    am_cols = []
    sm_cols = []
    lane = lax.broadcasted_iota(jnp.int32, (B, CW), 1)
    for c in range(CPT):
        seg = logits[:, c * CW:(c + 1) * CW]
        mx = jnp.max(seg, axis=1, keepdims=True)
        vocab0 = vt * VT + c * CW
        am = jnp.min(jnp.where(seg == mx, lane + vocab0, V),
                     axis=1, keepdims=True)
        sm = jnp.sum(jnp.exp(seg - mx), axis=1, keepdims=True)
        mx_cols.append(mx)
        am_cols.append(am)
        sm_cols.append(sm)
    cmax_ref[0, 0] = jnp.concatenate(mx_cols, axis=1)
    cam_ref[0, 0] = jnp.concatenate(am_cols, axis=1)
    csum_ref[0, 0] = jnp.concatenate(sm_cols, axis=1)


def _tc_phase(x, W_res, b_res, W_lm):
    return pl.pallas_call(
        _tc_body,
        grid=(HEADS, NVT),
        in_specs=[
            pl.BlockSpec((B, H), lambda h, v: (0, 0)),
            pl.BlockSpec((1, H, H), lambda h, v: (h, 0, 0)),
            pl.BlockSpec((1, 1, H), lambda h, v: (h, 0, 0)),
            pl.BlockSpec((1, VT, H), lambda h, v: (h, v, 0)),
        ],
        out_specs=[
            pl.BlockSpec((1, 1, B, CPT), lambda h, v: (h, v, 0, 0)),
            pl.BlockSpec((1, 1, B, CPT), lambda h, v: (h, v, 0, 0)),
            pl.BlockSpec((1, 1, B, CPT), lambda h, v: (h, v, 0, 0)),
            pl.BlockSpec((1, B, H), lambda h, v: (h, 0, 0)),
        ],
        out_shape=[
            jax.ShapeDtypeStruct((HEADS, NVT, B, CPT), jnp.float32),
            jax.ShapeDtypeStruct((HEADS, NVT, B, CPT), jnp.int32),
            jax.ShapeDtypeStruct((HEADS, NVT, B, CPT), jnp.float32),
            jax.ShapeDtypeStruct((HEADS, B, H), jnp.float32),
        ],
        scratch_shapes=[pltpu.VMEM((B, H), jnp.float32)],
    )(x, W_res, b_res.reshape(HEADS, 1, H), W_lm)


def _sc_phase(cmax, cam, csum, h_all, wlm_rows):
    mesh = plsc.VectorSubcoreMesh(core_axis_name="c", subcore_axis_name="s")

    def _lane_max(v):
        m = v[0]
        for i in range(1, 16):
            m = jnp.maximum(m, v[i])
        return m

    def _lane_sum(v):
        s = v[0]
        for i in range(1, 16):
            s = s + v[i]
        return s

    @functools.partial(
        pl.kernel,
        mesh=mesh,
        compiler_params=pltpu.CompilerParams(needs_layout_passes=False),
        out_type=[
            jax.ShapeDtypeStruct((NROWS, 16), jnp.float32),
            jax.ShapeDtypeStruct((NROWS, 16), jnp.int32),
        ],
        scratch_types=[
            pltpu.VMEM((RPW, NCHP), jnp.float32),
            pltpu.VMEM((RPW, NCHP), jnp.int32),
            pltpu.VMEM((RPW, NCHP), jnp.float32),
            pltpu.VMEM((48,), jnp.float32),
            pltpu.VMEM((48,), jnp.int32),
            pltpu.VMEM((32,), jnp.int32),
            pltpu.VMEM((32,), jnp.int32),
            pltpu.VMEM((8, 128), jnp.float32),
            pltpu.VMEM((128, 128), jnp.float32),
            pltpu.VMEM((RPW, 16), jnp.float32),
            pltpu.VMEM((RPW, 16), jnp.int32),
            pltpu.SMEM((4,), jnp.int32),
        ],
    )
    def k(cmax_hbm, cam_hbm, csum_hbm, h_hbm, wlm_hbm, vals_hbm, idx_hbm,
          cmax_v, cam_v, csum_v, cval_v, cidx_v, fw_v, fam_v, hrow_v, wblk_v,
          ov_v, oi_v, cnt_s):
        wid = lax.axis_index("s") * 2 + lax.axis_index("c")
        base = wid * RPW
        pltpu.sync_copy(cmax_hbm.at[pl.ds(base, RPW)], cmax_v)
        pltpu.sync_copy(cam_hbm.at[pl.ds(base, RPW)], cam_v)
        pltpu.sync_copy(csum_hbm.at[pl.ds(base, RPW)], csum_v)

        def row_body(r, carry):
            gr = base + r
            head = gr // B
            mvec = jnp.full((16,), NEG, jnp.float32)
            for j in range(NCHP // 16):
                mvec = jnp.maximum(mvec, cmax_v[r, pl.ds(j * 16, 16)])
            m = _lane_max(mvec)
            zvec = jnp.zeros((16,), jnp.float32)
            for j in range(NCHP // 16):
                cm = cmax_v[r, pl.ds(j * 16, 16)]
                cs = csum_v[r, pl.ds(j * 16, 16)]
                zvec = zvec + jnp.exp(cm - m) * cs
            z = _lane_sum(zvec)
            t = THRESH * z
            for j in range(3):
                cval_v[pl.ds(j * 16, 16)] = jnp.zeros((16,), jnp.float32)
                cidx_v[pl.ds(j * 16, 16)] = jnp.full((16,), -1, jnp.int32)
            cnt_s[0] = 0
            cnt_s[1] = 0
            for j in range(NCHP // 16):
                cm = cmax_v[r, pl.ds(j * 16, 16)]
                cs = csum_v[r, pl.ds(j * 16, 16)]
                am = cam_v[r, pl.ds(j * 16, 16)]
                e = jnp.exp(cm - m)
                sel = (e >= t) | (cm == m)
                cc = cnt_s[0]
                pref = plsc.cumsum(sel.astype(jnp.int32))
                dest = jnp.minimum(jnp.where(sel, cc + pref - 1, 47), 47)
                plsc.store_scatter(cval_v, [dest], e)
                plsc.store_scatter(cidx_v, [dest], am)
                cnt_s[0] = cc + pref[15]
                flag = sel & ((cs - 1.0) * e >= t)
                fb = head * V + (j * 16 + lax.iota(jnp.int32, 16)) * CW
                fc = cnt_s[1]
                fpref = plsc.cumsum(flag.astype(jnp.int32))
                fdest = jnp.minimum(jnp.where(flag, fc + fpref - 1, 31), 31)
                plsc.store_scatter(fw_v, [fdest], fb)
                plsc.store_scatter(fam_v, [fdest], am)
                cnt_s[1] = fc + fpref[15]
            nflag = jnp.minimum(cnt_s[1], 16)

            @pl.when(nflag > 0)
            def _():
                pltpu.sync_copy(
                    h_hbm.at[pl.ds(pl.multiple_of(gr * 8, 8), 8)], hrow_v)

                def flag_body(f, fcarry):
                    wb_vec = fw_v[pl.ds(f, 16)]
                    wb = wb_vec[0]
                    amv = fam_v[pl.ds(f, 16)]
                    am0 = amv[0]

                    def sub_body(sb, scarry):
                        off = pl.multiple_of(wb * 8 + sb * 128, 8)
                        pltpu.sync_copy(wlm_hbm.at[pl.ds(off, 128)], wblk_v)
                        dots = jnp.zeros((16,), jnp.float32)
                        for w in range(16):
                            acc = jnp.zeros((16,), jnp.float32)
                            for q in range(H // 16):
                                acc = acc + (hrow_v[q // 8,
                                                    pl.ds((q % 8) * 16, 16)]
                                             * wblk_v[w * 8 + q // 8,
                                                      pl.ds((q % 8) * 16, 16)])
                            dv = _lane_sum(acc)
                            dots = jnp.where(lax.iota(jnp.int32, 16) == w,
                                             dv, dots)
                        vid = (wb - head * V) + sb * 16 + lax.iota(
                            jnp.int32, 16)
                        ev = jnp.exp(dots - m)
                        cand = (ev >= t) & (vid != am0)
                        cc = cnt_s[0]
                        pref = plsc.cumsum(cand.astype(jnp.int32))
                        dest = jnp.minimum(
                            jnp.where(cand, cc + pref - 1, 47), 47)
                        plsc.store_scatter(cval_v, [dest], ev)
                        plsc.store_scatter(cidx_v, [dest], vid)
                        cnt_s[0] = cc + pref[15]
                        return scarry

                    lax.fori_loop(0, CW // 16, sub_body, 0)
                    return fcarry

                lax.fori_loop(0, nflag, flag_body, 0)

            ev16 = cval_v[pl.ds(0, 16)]
            iv16 = cidx_v[pl.ds(0, 16)]
            sv, si = plsc.sort_key_val(ev16, iv16, descending=True)
            ov_v[r, pl.ds(0, 16)] = sv / z
            oi_v[r, pl.ds(0, 16)] = si
            return carry

        lax.fori_loop(0, RPW, row_body, 0)
        pltpu.sync_copy(ov_v, vals_hbm.at[pl.ds(base, RPW)])
        pltpu.sync_copy(oi_v, idx_hbm.at[pl.ds(base, RPW)])

    return k(cmax, cam, csum, h_all, wlm_rows)


def kernel(hidden_states, W_res, b_res, W_lm, k):
    cmax4, cam4, csum4, h_all = _tc_phase(hidden_states, W_res, b_res, W_lm)
    def _rows(a4, pad):
        a = a4.transpose(0, 2, 1, 3).reshape(NROWS, NCH)
        return jnp.concatenate(
            [a, jnp.full((NROWS, NCHP - NCH), pad, a.dtype)], axis=1)

    cmax = _rows(cmax4, NEG)
    cam = _rows(cam4, 0)
    csum = _rows(csum4, 0.0)
    vals, idx = _sc_phase(cmax, cam, csum,
                          h_all.reshape(NROWS * 8, 128),
                          W_lm.reshape(HEADS * V * 8, 128))
    vals3 = vals.reshape(HEADS, B, 16)[:, :, :10]
    idx3 = idx.reshape(HEADS, B, 16)[:, :, :10]
    pos = jnp.arange(10)[None, None, :]
    keep = ((vals3 >= THRESH) | (pos == 0)) & (pos < k)
    return jnp.where(keep, vals3, 0.0), jnp.where(keep, idx3, -1)


# speedup vs baseline: 3.3323x; 3.3323x over previous
import functools

import jax
import jax.numpy as jnp
from jax import lax
from jax.experimental import pallas as pl
from jax.experimental.pallas import tpu as pltpu
from jax.experimental.pallas import tpu_sc as plsc

HEADS = 4
H = 1024
V = 32000
B = 128
THRESH = 0.09

VT = 3200
NVT = V // VT
CW = 128
CPT = VT // CW
NCH = V // CW
NCHP = 256
NROWS = HEADS * B
NWORK = 32
RPW = NROWS // NWORK
NEG = -3.0e38


def _tc_body(x_ref, wres_ref, bres_ref, wlm_ref,
             cmax_ref, cam_ref, csum_ref, h_out_ref, h_ref):
    vt = pl.program_id(1)

    @pl.when(vt == 0)
    def _():
        x = x_ref[...]
        pre = lax.dot_general(x, wres_ref[0], (((1,), (1,)), ((), ())),
                              preferred_element_type=jnp.float32)
        pre = pre + bres_ref[0]
        hv = x + pre * jax.nn.sigmoid(pre)
        h_ref[...] = hv
        h_out_ref[0] = hv

    logits = lax.dot_general(h_ref[...], wlm_ref[0], (((1,), (1,)), ((), ())),
                             preferred_element_type=jnp.float32)

    mx_cols = []
    am_cols = []
    sm_cols = []
    lane = lax.broadcasted_iota(jnp.int32, (B, CW), 1)
    for c in range(CPT):
        seg = logits[:, c * CW:(c + 1) * CW]
        mx = jnp.max(seg, axis=1, keepdims=True)
        vocab0 = vt * VT + c * CW
        am = jnp.min(jnp.where(seg == mx, lane + vocab0, V),
                     axis=1, keepdims=True)
        sm = jnp.sum(jnp.exp(seg - mx), axis=1, keepdims=True)
        mx_cols.append(mx)
        am_cols.append(am)
        sm_cols.append(sm)
    cmax_ref[0, 0] = jnp.concatenate(mx_cols, axis=1)
    cam_ref[0, 0] = jnp.concatenate(am_cols, axis=1)
    csum_ref[0, 0] = jnp.concatenate(sm_cols, axis=1)


def _tc_phase(x, W_res, b_res, W_lm):
    return pl.pallas_call(
        _tc_body,
        grid=(HEADS, NVT),
        in_specs=[
            pl.BlockSpec((B, H), lambda h, v: (0, 0)),
            pl.BlockSpec((1, H, H), lambda h, v: (h, 0, 0)),
            pl.BlockSpec((1, 1, H), lambda h, v: (h, 0, 0)),
            pl.BlockSpec((1, VT, H), lambda h, v: (h, v, 0)),
        ],
        out_specs=[
            pl.BlockSpec((1, 1, B, CPT), lambda h, v: (h, v, 0, 0)),
            pl.BlockSpec((1, 1, B, CPT), lambda h, v: (h, v, 0, 0)),
            pl.BlockSpec((1, 1, B, CPT), lambda h, v: (h, v, 0, 0)),
            pl.BlockSpec((1, B, H), lambda h, v: (h, 0, 0)),
        ],
        out_shape=[
            jax.ShapeDtypeStruct((HEADS, NVT, B, CPT), jnp.float32),
            jax.ShapeDtypeStruct((HEADS, NVT, B, CPT), jnp.int32),
            jax.ShapeDtypeStruct((HEADS, NVT, B, CPT), jnp.float32),
            jax.ShapeDtypeStruct((HEADS, B, H), jnp.float32),
        ],
        scratch_shapes=[pltpu.VMEM((B, H), jnp.float32)],
    )(x, W_res, b_res.reshape(HEADS, 1, H), W_lm)


def _sc_phase(cmax, cam, csum, h_all, wlm_rows):
    mesh = plsc.VectorSubcoreMesh(core_axis_name="c", subcore_axis_name="s")

    def _lane_max(v):
        m = v[0]
        for i in range(1, 16):
            m = jnp.maximum(m, v[i])
        return m

    def _lane_sum(v):
        s = v[0]
        for i in range(1, 16):
            s = s + v[i]
        return s

    @functools.partial(
        pl.kernel,
        mesh=mesh,
        compiler_params=pltpu.CompilerParams(needs_layout_passes=False),
        out_type=[
            jax.ShapeDtypeStruct((NROWS, 16), jnp.float32),
            jax.ShapeDtypeStruct((NROWS, 16), jnp.int32),
        ],
        scratch_types=[
            pltpu.VMEM((RPW, NCHP), jnp.float32),
            pltpu.VMEM((RPW, NCHP), jnp.int32),
            pltpu.VMEM((RPW, NCHP), jnp.float32),
            pltpu.VMEM((48,), jnp.float32),
            pltpu.VMEM((48,), jnp.int32),
            pltpu.VMEM((32,), jnp.int32),
            pltpu.VMEM((32,), jnp.int32),
            pltpu.VMEM((8, H), jnp.float32),
            pltpu.VMEM((16, H), jnp.float32),
            pltpu.VMEM((RPW, 16), jnp.float32),
            pltpu.VMEM((RPW, 16), jnp.int32),
            pltpu.SMEM((4,), jnp.int32),
        ],
    )
    def k(cmax_hbm, cam_hbm, csum_hbm, h_hbm, wlm_hbm, vals_hbm, idx_hbm,
          cmax_v, cam_v, csum_v, cval_v, cidx_v, fw_v, fam_v, hrow_v, wblk_v,
          ov_v, oi_v, cnt_s):
        wid = lax.axis_index("s") * 2 + lax.axis_index("c")
        base = wid * RPW
        pltpu.sync_copy(cmax_hbm.at[pl.ds(base, RPW)], cmax_v)
        pltpu.sync_copy(cam_hbm.at[pl.ds(base, RPW)], cam_v)
        pltpu.sync_copy(csum_hbm.at[pl.ds(base, RPW)], csum_v)

        def row_body(r, carry):
            gr = base + r
            head = gr // B
            mvec = jnp.full((16,), NEG, jnp.float32)
            for j in range(NCHP // 16):
                mvec = jnp.maximum(mvec, cmax_v[r, pl.ds(j * 16, 16)])
            m = _lane_max(mvec)
            zvec = jnp.zeros((16,), jnp.float32)
            for j in range(NCHP // 16):
                cm = cmax_v[r, pl.ds(j * 16, 16)]
                cs = csum_v[r, pl.ds(j * 16, 16)]
                zvec = zvec + jnp.exp(cm - m) * cs
            z = _lane_sum(zvec)
            t = THRESH * z
            for j in range(3):
                cval_v[pl.ds(j * 16, 16)] = jnp.zeros((16,), jnp.float32)
                cidx_v[pl.ds(j * 16, 16)] = jnp.full((16,), -1, jnp.int32)
            cnt_s[0] = 0
            cnt_s[1] = 0
            for j in range(NCHP // 16):
                cm = cmax_v[r, pl.ds(j * 16, 16)]
                cs = csum_v[r, pl.ds(j * 16, 16)]
                am = cam_v[r, pl.ds(j * 16, 16)]
                e = jnp.exp(cm - m)
                sel = (e >= t) | (cm == m)
                cc = cnt_s[0]
                pref = plsc.cumsum(sel.astype(jnp.int32))
                dest = jnp.minimum(jnp.where(sel, cc + pref - 1, 47), 47)
                plsc.store_scatter(cval_v, [dest], e)
                plsc.store_scatter(cidx_v, [dest], am)
                cnt_s[0] = cc + pref[15]
                flag = sel & ((cs - 1.0) * e >= t)
                fb = (j * 16 + lax.iota(jnp.int32, 16)) * CW
                fc = cnt_s[1]
                fpref = plsc.cumsum(flag.astype(jnp.int32))
                fdest = jnp.minimum(jnp.where(flag, fc + fpref - 1, 31), 31)
                plsc.store_scatter(fw_v, [fdest], fb)
                plsc.store_scatter(fam_v, [fdest], am)
                cnt_s[1] = fc + fpref[15]
            nflag = jnp.minimum(cnt_s[1], 16)

            @pl.when(nflag > 0)
            def _():
                rr = gr - head * B
                rr8 = (rr // 8) * 8
                hsel = rr - rr8
                pltpu.sync_copy(
                    h_hbm.at[head, pl.ds(pl.multiple_of(rr8, 8), 8)], hrow_v)

                def flag_body(f, fcarry):
                    wb_vec = fw_v[pl.ds(f, 16)]
                    vb = wb_vec[0]
                    amv = fam_v[pl.ds(f, 16)]
                    am0 = amv[0]

                    def sub_body(sb, scarry):
                        off = pl.multiple_of(vb + sb * 16, 8)
                        pltpu.sync_copy(wlm_hbm.at[head, pl.ds(off, 16)],
                                        wblk_v)
                        dots = jnp.zeros((16,), jnp.float32)
                        for w in range(16):
                            acc = jnp.zeros((16,), jnp.float32)
                            for q in range(H // 16):
                                acc = acc + (hrow_v[hsel, pl.ds(q * 16, 16)]
                                             * wblk_v[w, pl.ds(q * 16, 16)])
                            dv = _lane_sum(acc)
                            dots = jnp.where(lax.iota(jnp.int32, 16) == w,
                                             dv, dots)
                        vid = vb + sb * 16 + lax.iota(jnp.int32, 16)
                        ev = jnp.exp(dots - m)
                        cand = (ev >= t) & (vid != am0)
                        cc = cnt_s[0]
                        pref = plsc.cumsum(cand.astype(jnp.int32))
                        dest = jnp.minimum(
                            jnp.where(cand, cc + pref - 1, 47), 47)
                        plsc.store_scatter(cval_v, [dest], ev)
                        plsc.store_scatter(cidx_v, [dest], vid)
                        cnt_s[0] = cc + pref[15]
                        return scarry

                    lax.fori_loop(0, CW // 16, sub_body, 0)
                    return fcarry

                lax.fori_loop(0, nflag, flag_body, 0)

            ev16 = cval_v[pl.ds(0, 16)]
            iv16 = cidx_v[pl.ds(0, 16)]
            sv, si = plsc.sort_key_val(ev16, iv16, descending=True)
            ov_v[r, pl.ds(0, 16)] = sv / z
            oi_v[r, pl.ds(0, 16)] = si
            return carry

        lax.fori_loop(0, RPW, row_body, 0)
        pltpu.sync_copy(ov_v, vals_hbm.at[pl.ds(base, RPW)])
        pltpu.sync_copy(oi_v, idx_hbm.at[pl.ds(base, RPW)])

    return k(cmax, cam, csum, h_all, wlm_rows)


def kernel(hidden_states, W_res, b_res, W_lm, k):
    cmax4, cam4, csum4, h_all = _tc_phase(hidden_states, W_res, b_res, W_lm)
    def _rows(a4, pad):
        a = a4.transpose(0, 2, 1, 3).reshape(NROWS, NCH)
        return jnp.concatenate(
            [a, jnp.full((NROWS, NCHP - NCH), pad, a.dtype)], axis=1)

    cmax = _rows(cmax4, NEG)
    cam = _rows(cam4, 0)
    csum = _rows(csum4, 0.0)
    vals, idx = _sc_phase(cmax, cam, csum, h_all, W_lm)
    vals3 = vals.reshape(HEADS, B, 16)[:, :, :10]
    idx3 = idx.reshape(HEADS, B, 16)[:, :, :10]
    pos = jnp.arange(10)[None, None, :]
    keep = ((vals3 >= THRESH) | (pos == 0)) & (pos < k)
    return jnp.where(keep, vals3, 0.0), jnp.where(keep, idx3, -1)
